# Initial kernel scaffold; baseline (speedup 1.0000x reference)
import jax
import jax.numpy as jnp
from jax.experimental import pallas as pl


def kernel(labela, labelb, pred_ab, pred_ba):
    raise NotImplementedError("write your pallas kernel here")


# speedup vs baseline: 1.3596x; 1.3596x over previous
import jax
import jax.numpy as jnp
from jax.experimental import pallas as pl
from jax.experimental.pallas import tpu as pltpu

_B, _C, _H, _W = 128, 5, 128, 128
_BB = 8
_G = _B // _BB


def _loss_kernel(a_ref, b_ref, pab_ref, pba_ref, o_ref):
    a = a_ref[...]
    d1 = a - pab_ref[...]
    d2 = b_ref[...] - pba_ref[...]
    s = d1 * d1 + d2 * d2
    per_cell = jnp.sum(s, axis=1)
    mask = a[:, 0] != 0
    masked = jnp.where(mask, per_cell, 0.0)
    o_ref[0, 0, :] = jnp.sum(masked, axis=(0, 1))


def kernel(labela, labelb, pred_ab, pred_ba):
    in_spec = pl.BlockSpec((_BB, _C, _H, _W), lambda i: (i, 0, 0, 0))
    partials = pl.pallas_call(
        _loss_kernel,
        out_shape=jax.ShapeDtypeStruct((_G, 1, _W), jnp.float32),
        grid=(_G,),
        in_specs=[in_spec, in_spec, in_spec, in_spec],
        out_specs=pl.BlockSpec((1, 1, _W), lambda i: (i, 0, 0)),
        compiler_params=pltpu.CompilerParams(
            dimension_semantics=("parallel",),
        ),
        name="yolo_masked_sse",
    )(labela, labelb, pred_ab, pred_ba)
    return jnp.sum(partials)


# speedup vs baseline: 1.3831x; 1.0172x over previous
import jax
import jax.numpy as jnp
from jax.experimental import pallas as pl
from jax.experimental.pallas import tpu as pltpu

_B, _C, _H, _W = 128, 5, 128, 128
_BB = 16
_G = _B // _BB


def _loss_kernel(a_ref, b_ref, pab_ref, pba_ref, o_ref):
    a = a_ref[...]
    d1 = a - pab_ref[...]
    d2 = b_ref[...] - pba_ref[...]
    s = d1 * d1 + d2 * d2
    per_cell = jnp.sum(s, axis=1)
    mask = a[:, 0] != 0
    masked = jnp.where(mask, per_cell, 0.0)
    o_ref[0, 0, :] = jnp.sum(masked, axis=(0, 1))


def kernel(labela, labelb, pred_ab, pred_ba):
    in_spec = pl.BlockSpec((_BB, _C, _H, _W), lambda i: (i, 0, 0, 0))
    partials = pl.pallas_call(
        _loss_kernel,
        out_shape=jax.ShapeDtypeStruct((_G, 1, _W), jnp.float32),
        grid=(_G,),
        in_specs=[in_spec, in_spec, in_spec, in_spec],
        out_specs=pl.BlockSpec((1, 1, _W), lambda i: (i, 0, 0)),
        compiler_params=pltpu.CompilerParams(
            dimension_semantics=("parallel",),
            vmem_limit_bytes=50 * 1024 * 1024,
        ),
        name="yolo_masked_sse",
    )(labela, labelb, pred_ab, pred_ba)
    return jnp.sum(partials)


# speedup vs baseline: 1.4490x; 1.0477x over previous
import jax
import jax.numpy as jnp
from jax.experimental import pallas as pl
from jax.experimental.pallas import tpu as pltpu

_B, _C, _H, _W = 128, 5, 128, 128
_BB = 16
_G = _B // _BB


def _loss_kernel(a_ref, b_ref, pab_ref, pba_ref, o_ref):
    acc2d = jnp.zeros((_H, _W), jnp.float32)
    for i in range(_BB):
        cell = None
        for c in range(_C):
            d1 = a_ref[i, c] - pab_ref[i, c]
            d2 = b_ref[i, c] - pba_ref[i, c]
            t = d1 * d1 + d2 * d2
            cell = t if cell is None else cell + t
        acc2d = acc2d + jnp.where(a_ref[i, 0] != 0, cell, 0.0)
    o_ref[0, 0, :] = jnp.sum(acc2d, axis=0)


def kernel(labela, labelb, pred_ab, pred_ba):
    in_spec = pl.BlockSpec((_BB, _C, _H, _W), lambda i: (i, 0, 0, 0))
    partials = pl.pallas_call(
        _loss_kernel,
        out_shape=jax.ShapeDtypeStruct((_G, 1, _W), jnp.float32),
        grid=(_G,),
        in_specs=[in_spec, in_spec, in_spec, in_spec],
        out_specs=pl.BlockSpec((1, 1, _W), lambda i: (i, 0, 0)),
        compiler_params=pltpu.CompilerParams(
            dimension_semantics=("parallel",),
            vmem_limit_bytes=50 * 1024 * 1024,
        ),
        name="yolo_masked_sse",
    )(labela, labelb, pred_ab, pred_ba)
    return jnp.sum(partials)


# speedup vs baseline: 1.4876x; 1.0267x over previous
import jax
import jax.numpy as jnp
from jax.experimental import pallas as pl
from jax.experimental.pallas import tpu as pltpu

_B, _C, _H, _W = 128, 5, 128, 128
_BB = 8
_G = _B // _BB


def _loss_kernel(a_ref, b_ref, pab_ref, pba_ref, o_ref):
    acc2d = jnp.zeros((_H, _W), jnp.float32)
    for i in range(_BB):
        cell = None
        for c in range(_C):
            d1 = a_ref[i, c] - pab_ref[i, c]
            d2 = b_ref[i, c] - pba_ref[i, c]
            t = d1 * d1 + d2 * d2
            cell = t if cell is None else cell + t
        acc2d = acc2d + jnp.where(a_ref[i, 0] != 0, cell, 0.0)
    o_ref[0, 0, :] = jnp.sum(acc2d, axis=0)


def kernel(labela, labelb, pred_ab, pred_ba):
    in_spec = pl.BlockSpec((_BB, _C, _H, _W), lambda i: (i, 0, 0, 0))
    partials = pl.pallas_call(
        _loss_kernel,
        out_shape=jax.ShapeDtypeStruct((_G, 1, _W), jnp.float32),
        grid=(_G,),
        in_specs=[in_spec, in_spec, in_spec, in_spec],
        out_specs=pl.BlockSpec((1, 1, _W), lambda i: (i, 0, 0)),
        compiler_params=pltpu.CompilerParams(
            dimension_semantics=("parallel",),
            vmem_limit_bytes=50 * 1024 * 1024,
        ),
        name="yolo_masked_sse",
    )(labela, labelb, pred_ab, pred_ba)
    return jnp.sum(partials)
